# baseline (device time: 26736 ns/iter reference)
import jax
import jax.numpy as jnp
from jax import lax
from jax.experimental import pallas as pl
from jax.experimental.pallas import tpu as pltpu

N_DEV = 4
B_LOC = 2
SQ = 128
D = 512
H_LOC = 8
DH = 64
SCALE = 0.125
BH = B_LOC * H_LOC


def kernel(x, Wq, Wo, Wk, Wv):
    def body(x_ref, wq_ref, wo_ref, wk_ref, wv_ref, out_ref,
             comm_ref, part_ref, rsbuf_ref, wqkv_ref,
             q4_ref, k4_ref, v4_ref, attn_ref,
             ag_send, ag_recv, rs_send, rs_recv):
        my = lax.axis_index("i")
        left = (my + N_DEV - 1) % N_DEV
        right = (my + 1) % N_DEV
        diag = (my + 2) % N_DEV

        barrier = pltpu.get_barrier_semaphore()
        for nbr in (left, right, diag):
            pl.semaphore_signal(barrier, inc=1, device_id=(nbr,),
                                device_id_type=pl.DeviceIdType.MESH)

        def ag_send_to(dst_dev, slot, sem_i):
            return pltpu.make_async_remote_copy(
                src_ref=comm_ref.at[0],
                dst_ref=comm_ref.at[slot],
                send_sem=ag_send.at[sem_i],
                recv_sem=ag_recv.at[slot],
                device_id=(dst_dev,),
                device_id_type=pl.DeviceIdType.MESH,
            )

        def ag_recv_from(src_dev, slot):
            return pltpu.make_async_remote_copy(
                src_ref=comm_ref.at[0],
                dst_ref=comm_ref.at[slot],
                send_sem=ag_send.at[0],
                recv_sem=ag_recv.at[slot],
                device_id=(src_dev,),
                device_id_type=pl.DeviceIdType.MESH,
            )

        def rs_send_to(dst_dev, part_slot, buf_slot, sem_i=None):
            sem_i = buf_slot if sem_i is None else sem_i
            return pltpu.make_async_remote_copy(
                src_ref=part_ref.at[part_slot],
                dst_ref=rsbuf_ref.at[buf_slot],
                send_sem=rs_send.at[sem_i],
                recv_sem=rs_recv.at[sem_i],
                device_id=(dst_dev,),
                device_id_type=pl.DeviceIdType.MESH,
            )

        def rs_send_half(dst_dev, part_slot, buf_slot, b, sem_i):
            return pltpu.make_async_remote_copy(
                src_ref=part_ref.at[part_slot, b],
                dst_ref=rsbuf_ref.at[buf_slot, b],
                send_sem=rs_send.at[sem_i],
                recv_sem=rs_recv.at[sem_i],
                device_id=(dst_dev,),
                device_id_type=pl.DeviceIdType.MESH,
            )

        def rs_recv_from(src_dev, buf_slot, sem_i=None, b=None):
            sem_i = buf_slot if sem_i is None else sem_i
            dst = (rsbuf_ref.at[buf_slot] if b is None
                   else rsbuf_ref.at[buf_slot, b])
            return pltpu.make_async_remote_copy(
                src_ref=part_ref.at[0] if b is None else part_ref.at[0, 0],
                dst_ref=dst,
                send_sem=rs_send.at[0],
                recv_sem=rs_recv.at[sem_i],
                device_id=(src_dev,),
                device_id_type=pl.DeviceIdType.MESH,
            )

        def compute_batches(r, batches):
            nb = len(batches)
            b0 = batches[0]
            x2 = comm_ref[r, b0:b0 + nb].reshape(nb * SQ, D)
            qkv = lax.dot(x2, wqkv_ref[...],
                          preferred_element_type=jnp.float32
                          ).astype(jnp.bfloat16)
            for j in range(nb):
                rows = slice(j * SQ, (j + 1) * SQ)
                for h in range(H_LOC):
                    i = j * H_LOC + h
                    q4_ref[i] = qkv[rows, 0 * D + h * DH:0 * D + (h + 1) * DH]
                    k4_ref[i] = qkv[rows, 1 * D + h * DH:1 * D + (h + 1) * DH]
                    v4_ref[i] = qkv[rows, 2 * D + h * DH:2 * D + (h + 1) * DH]
            s = lax.dot_general(
                q4_ref[0:nb * H_LOC], k4_ref[0:nb * H_LOC],
                dimension_numbers=(((2,), (2,)), ((0,), (0,))),
                preferred_element_type=jnp.float32,
            )
            e = jnp.exp(s)
            l = jnp.sum(e, axis=2, keepdims=True)
            o = lax.dot_general(
                e.astype(jnp.bfloat16), v4_ref[0:nb * H_LOC],
                dimension_numbers=(((2,), (1,)), ((0,), (0,))),
                preferred_element_type=jnp.float32,
            )
            ob = (o * (1.0 / l)).astype(jnp.bfloat16)
            for j in range(nb):
                rows = slice(j * SQ, (j + 1) * SQ)
                for h in range(H_LOC):
                    attn_ref[rows, h * DH:(h + 1) * DH] = ob[j * H_LOC + h]
            part = lax.dot(
                attn_ref[0:nb * SQ], wo, preferred_element_type=jnp.float32
            ).astype(jnp.bfloat16)
            for j, b in enumerate(batches):
                part_ref[r, b] = part[j * SQ:(j + 1) * SQ]

        def compute_chunk(r):
            compute_batches(r, list(range(B_LOC)))

        comm_ref[0] = x_ref[...].astype(jnp.bfloat16)
        wqkv_ref[:, 0 * D:1 * D] = (wq_ref[...] * SCALE).astype(jnp.bfloat16)
        wqkv_ref[:, 1 * D:2 * D] = wk_ref[...].astype(jnp.bfloat16)
        wqkv_ref[:, 2 * D:3 * D] = wv_ref[...].astype(jnp.bfloat16)
        pl.semaphore_wait(barrier, 3)
        snd_l = ag_send_to(left, 1, 0)
        snd_r = ag_send_to(right, 3, 1)
        snd_d = ag_send_to(diag, 2, 2)
        snd_l.start()
        snd_r.start()
        snd_d.start()

        wo = wo_ref[...].astype(jnp.bfloat16)
        compute_chunk(0)

        ag_recv_from(right, 1).wait_recv()
        compute_chunk(1)
        rs_r = rs_send_to(right, 1, 0)
        rs_r.start()

        ag_recv_from(left, 3).wait_recv()
        compute_chunk(3)
        rs_l = rs_send_to(left, 3, 1)
        rs_l.start()

        ag_recv_from(diag, 2).wait_recv()
        compute_chunk(2)
        rs_d = rs_send_to(diag, 2, 2)
        rs_d.start()

        rs_recv_from(left, 0).wait_recv()
        acc = (part_ref[0].astype(jnp.float32)
               + rsbuf_ref[0].astype(jnp.float32))
        rs_recv_from(right, 1).wait_recv()
        acc = acc + rsbuf_ref[1].astype(jnp.float32)
        rs_recv_from(diag, 2).wait_recv()
        out_ref[...] = acc + rsbuf_ref[2].astype(jnp.float32)

        for snd in (snd_l, snd_r, snd_d, rs_r, rs_l, rs_d):
            snd.wait_send()

    return pl.pallas_call(
        body,
        out_shape=jax.ShapeDtypeStruct((B_LOC, SQ, D), jnp.float32),
        in_specs=[pl.BlockSpec(memory_space=pltpu.VMEM)] * 5,
        out_specs=pl.BlockSpec(memory_space=pltpu.VMEM),
        scratch_shapes=[
            pltpu.VMEM((N_DEV, B_LOC, SQ, D), jnp.bfloat16),
            pltpu.VMEM((N_DEV, B_LOC, SQ, D), jnp.bfloat16),
            pltpu.VMEM((3, B_LOC, SQ, D), jnp.bfloat16),
            pltpu.VMEM((D, 3 * D), jnp.bfloat16),
            pltpu.VMEM((BH, SQ, DH), jnp.bfloat16),
            pltpu.VMEM((BH, SQ, DH), jnp.bfloat16),
            pltpu.VMEM((BH, SQ, DH), jnp.bfloat16),
            pltpu.VMEM((B_LOC * SQ, D), jnp.bfloat16),
            pltpu.SemaphoreType.DMA((3,)),
            pltpu.SemaphoreType.DMA((N_DEV,)),
            pltpu.SemaphoreType.DMA((4,)),
            pltpu.SemaphoreType.DMA((4,)),
        ],
        compiler_params=pltpu.CompilerParams(collective_id=0),
    )(x, Wq, Wo, Wk, Wv)


# device time: 25995 ns/iter; 1.0285x vs baseline; 1.0285x over previous
import jax
import jax.numpy as jnp
from jax import lax
from jax.experimental import pallas as pl
from jax.experimental.pallas import tpu as pltpu

N_DEV = 4
B_LOC = 2
SQ = 128
D = 512
H_LOC = 8
DH = 64
SCALE = 0.125
BH = B_LOC * H_LOC


def kernel(x, Wq, Wo, Wk, Wv):
    def body(x_ref, wq_ref, wo_ref, wk_ref, wv_ref, out_ref,
             comm_ref, part_ref, rsbuf_ref, wqkv_ref,
             q4_ref, k4_ref, v4_ref, attn_ref,
             ag_send, ag_recv, rs_send, rs_recv):
        my = lax.axis_index("i")
        left = (my + N_DEV - 1) % N_DEV
        right = (my + 1) % N_DEV
        diag = (my + 2) % N_DEV

        barrier = pltpu.get_barrier_semaphore()
        for nbr in (left, right, diag):
            pl.semaphore_signal(barrier, inc=1, device_id=(nbr,),
                                device_id_type=pl.DeviceIdType.MESH)

        def ag_send_to(dst_dev, slot, sem_i):
            return pltpu.make_async_remote_copy(
                src_ref=comm_ref.at[0],
                dst_ref=comm_ref.at[slot],
                send_sem=ag_send.at[sem_i],
                recv_sem=ag_recv.at[slot],
                device_id=(dst_dev,),
                device_id_type=pl.DeviceIdType.MESH,
            )

        def ag_recv_from(src_dev, slot):
            return pltpu.make_async_remote_copy(
                src_ref=comm_ref.at[0],
                dst_ref=comm_ref.at[slot],
                send_sem=ag_send.at[0],
                recv_sem=ag_recv.at[slot],
                device_id=(src_dev,),
                device_id_type=pl.DeviceIdType.MESH,
            )

        def rs_send_to(dst_dev, part_slot, buf_slot, sem_i=None):
            sem_i = buf_slot if sem_i is None else sem_i
            return pltpu.make_async_remote_copy(
                src_ref=part_ref.at[part_slot],
                dst_ref=rsbuf_ref.at[buf_slot],
                send_sem=rs_send.at[sem_i],
                recv_sem=rs_recv.at[sem_i],
                device_id=(dst_dev,),
                device_id_type=pl.DeviceIdType.MESH,
            )

        def rs_send_half(dst_dev, part_slot, buf_slot, b, sem_i):
            return pltpu.make_async_remote_copy(
                src_ref=part_ref.at[part_slot, b],
                dst_ref=rsbuf_ref.at[buf_slot, b],
                send_sem=rs_send.at[sem_i],
                recv_sem=rs_recv.at[sem_i],
                device_id=(dst_dev,),
                device_id_type=pl.DeviceIdType.MESH,
            )

        def rs_recv_from(src_dev, buf_slot, sem_i=None, b=None):
            sem_i = buf_slot if sem_i is None else sem_i
            dst = (rsbuf_ref.at[buf_slot] if b is None
                   else rsbuf_ref.at[buf_slot, b])
            return pltpu.make_async_remote_copy(
                src_ref=part_ref.at[0] if b is None else part_ref.at[0, 0],
                dst_ref=dst,
                send_sem=rs_send.at[0],
                recv_sem=rs_recv.at[sem_i],
                device_id=(src_dev,),
                device_id_type=pl.DeviceIdType.MESH,
            )

        def compute_batches(r, batches):
            nb = len(batches)
            b0 = batches[0]
            x2 = comm_ref[r, b0:b0 + nb].reshape(nb * SQ, D)
            qkv = lax.dot(x2, wqkv_ref[...],
                          preferred_element_type=jnp.float32
                          ).astype(jnp.bfloat16)
            for j in range(nb):
                rows = slice(j * SQ, (j + 1) * SQ)
                for h in range(H_LOC):
                    i = j * H_LOC + h
                    q4_ref[i] = qkv[rows, 0 * D + h * DH:0 * D + (h + 1) * DH]
                    k4_ref[i] = qkv[rows, 1 * D + h * DH:1 * D + (h + 1) * DH]
                    v4_ref[i] = qkv[rows, 2 * D + h * DH:2 * D + (h + 1) * DH]
            s = lax.dot_general(
                q4_ref[0:nb * H_LOC], k4_ref[0:nb * H_LOC],
                dimension_numbers=(((2,), (2,)), ((0,), (0,))),
                preferred_element_type=jnp.float32,
            )
            e = jnp.exp(s)
            l = jnp.sum(e, axis=2, keepdims=True)
            o = lax.dot_general(
                e.astype(jnp.bfloat16), v4_ref[0:nb * H_LOC],
                dimension_numbers=(((2,), (1,)), ((0,), (0,))),
                preferred_element_type=jnp.float32,
            )
            ob = (o * (1.0 / l)).astype(jnp.bfloat16)
            for j in range(nb):
                rows = slice(j * SQ, (j + 1) * SQ)
                for h in range(H_LOC):
                    attn_ref[rows, h * DH:(h + 1) * DH] = ob[j * H_LOC + h]
            part = lax.dot(
                attn_ref[0:nb * SQ], wo, preferred_element_type=jnp.float32
            ).astype(jnp.bfloat16)
            for j, b in enumerate(batches):
                part_ref[r, b] = part[j * SQ:(j + 1) * SQ]

        def compute_chunk(r):
            compute_batches(r, list(range(B_LOC)))

        comm_ref[0] = x_ref[...].astype(jnp.bfloat16)
        wqkv_ref[:, 0 * D:1 * D] = (wq_ref[...] * SCALE).astype(jnp.bfloat16)
        wqkv_ref[:, 1 * D:2 * D] = wk_ref[...].astype(jnp.bfloat16)
        wqkv_ref[:, 2 * D:3 * D] = wv_ref[...].astype(jnp.bfloat16)
        pl.semaphore_wait(barrier, 3)
        snd_l = ag_send_to(left, 1, 0)
        snd_r = ag_send_to(right, 3, 1)
        snd_d = ag_send_to(diag, 2, 2)
        snd_l.start()
        snd_r.start()
        snd_d.start()

        wo = wo_ref[...].astype(jnp.bfloat16)
        compute_chunk(0)

        ag_recv_from(left, 3).wait_recv()
        compute_chunk(3)
        rs_l = rs_send_to(left, 3, 1)
        rs_l.start()

        ag_recv_from(right, 1).wait_recv()
        compute_chunk(1)
        rs_r = rs_send_to(right, 1, 0)
        rs_r.start()

        ag_recv_from(diag, 2).wait_recv()
        compute_chunk(2)
        rs_d = rs_send_to(diag, 2, 2)
        rs_d.start()

        rs_recv_from(right, 1).wait_recv()
        acc = (part_ref[0].astype(jnp.float32)
               + rsbuf_ref[1].astype(jnp.float32))
        rs_recv_from(left, 0).wait_recv()
        acc = acc + rsbuf_ref[0].astype(jnp.float32)
        rs_recv_from(diag, 2).wait_recv()
        out_ref[...] = acc + rsbuf_ref[2].astype(jnp.float32)

        for snd in (snd_l, snd_r, snd_d, rs_r, rs_l, rs_d):
            snd.wait_send()

    return pl.pallas_call(
        body,
        out_shape=jax.ShapeDtypeStruct((B_LOC, SQ, D), jnp.float32),
        in_specs=[pl.BlockSpec(memory_space=pltpu.VMEM)] * 5,
        out_specs=pl.BlockSpec(memory_space=pltpu.VMEM),
        scratch_shapes=[
            pltpu.VMEM((N_DEV, B_LOC, SQ, D), jnp.bfloat16),
            pltpu.VMEM((N_DEV, B_LOC, SQ, D), jnp.bfloat16),
            pltpu.VMEM((3, B_LOC, SQ, D), jnp.bfloat16),
            pltpu.VMEM((D, 3 * D), jnp.bfloat16),
            pltpu.VMEM((BH, SQ, DH), jnp.bfloat16),
            pltpu.VMEM((BH, SQ, DH), jnp.bfloat16),
            pltpu.VMEM((BH, SQ, DH), jnp.bfloat16),
            pltpu.VMEM((B_LOC * SQ, D), jnp.bfloat16),
            pltpu.SemaphoreType.DMA((3,)),
            pltpu.SemaphoreType.DMA((N_DEV,)),
            pltpu.SemaphoreType.DMA((4,)),
            pltpu.SemaphoreType.DMA((4,)),
        ],
        compiler_params=pltpu.CompilerParams(collective_id=0),
    )(x, Wq, Wo, Wk, Wv)
